# baseline (speedup 1.0000x reference)
import functools

import jax
import jax.numpy as jnp
from jax import lax
from jax.experimental import pallas as pl
from jax.experimental.pallas import tpu as pltpu
from jax.experimental.pallas import tpu_sc as plsc

HIDDEN = 2048
EXPERTS = 64
TOPK = 8
EPS = 1e-6
BLOCK = 2048

NUM_CORES = 2
NUM_SUBCORES = 16
LANES = 16
NW = NUM_CORES * NUM_SUBCORES
MASKED = -2.0


def _probs_block(h_ref, w_ref, sv_ref, p_ref):
    h = h_ref[...]
    var = jnp.mean(h * h, axis=1, keepdims=True)
    hn = h * jax.lax.rsqrt(var + EPS)
    hn = hn * sv_ref[...]
    logits = jnp.dot(hn, w_ref[...], preferred_element_type=jnp.float32)
    mx = jnp.max(logits, axis=1, keepdims=True)
    ex = jnp.exp(logits - mx)
    p_ref[...] = ex / jnp.sum(ex, axis=1, keepdims=True)


def _sc_router(tpw, p_hbm, pes_hbm, w_hbm, i_hbm, c_hbm,
               pbuf, wbuf, ibuf, hbuf, pesbuf, cntbuf):
    wid = lax.axis_index("s") * NUM_CORES + lax.axis_index("c")
    base = wid * tpw
    lanes = lax.iota(jnp.int32, LANES)
    fzero = jnp.zeros((LANES,), jnp.float32)
    ones = jnp.ones((LANES,), jnp.float32)

    pltpu.sync_copy(p_hbm.at[pl.ds(base * EXPERTS, tpw * EXPERTS)], pbuf)
    pltpu.sync_copy(pes_hbm, pesbuf)
    for i in range(EXPERTS * LANES // LANES):
        hbuf[pl.ds(i * LANES, LANES)] = fzero

    def merge(am, ai, bm, bi):
        take = am >= bm
        return jnp.where(take, am, bm), jnp.where(take, ai, bi)

    def chunk_tree(idx_base, const_off):
        ms, is_ = [], []
        for j in range(0, LANES, 2):
            va = plsc.load_gather(pbuf, [idx_base + j])
            vb = plsc.load_gather(pbuf, [idx_base + (j + 1)])
            take = va >= vb
            ms.append(jnp.where(take, va, vb))
            is_.append(jnp.where(take,
                                 jnp.full((LANES,), j, jnp.int32),
                                 jnp.full((LANES,), j + 1, jnp.int32)))
        while len(ms) > 1:
            nm, ni = [], []
            for a in range(0, len(ms), 2):
                m2, i2 = merge(ms[a], is_[a], ms[a + 1], is_[a + 1])
                nm.append(m2)
                ni.append(i2)
            ms, is_ = nm, ni
        return ms[0], is_[0] + const_off

    def group_body(g, carry):
        rows = g * LANES + lanes
        rowbase = rows * EXPERTS
        outbase = rows * TOPK

        nchunks = EXPERTS // LANES
        cms, cis = [], []
        for c in range(nchunks):
            m, i = chunk_tree(rowbase + c * LANES,
                              jnp.full((LANES,), c * LANES, jnp.int32))
            cms.append(m)
            cis.append(i)

        def merge4():
            m01, i01 = merge(cms[0], cis[0], cms[1], cis[1])
            m23, i23 = merge(cms[2], cis[2], cms[3], cis[3])
            return merge(m01, i01, m23, i23)

        gm, gi = merge4()
        sel_w = []
        sel_i = []
        for r in range(TOPK):
            sel_w.append(gm)
            sel_i.append(gi)
            plsc.store_scatter(pbuf, [rowbase + gi],
                               jnp.full((LANES,), MASKED, jnp.float32))
            plsc.addupdate_scatter(hbuf, [lanes * EXPERTS + gi], ones)
            if r < TOPK - 1:
                cstar = jnp.right_shift(gi, 4)
                coff = jnp.left_shift(cstar, 4)
                nm, ni = chunk_tree(rowbase + coff, coff)
                for c in range(nchunks):
                    hit = cstar == c
                    cms[c] = jnp.where(hit, nm, cms[c])
                    cis[c] = jnp.where(hit, ni, cis[c])
                gm, gi = merge4()
        s = sel_w[0]
        for r in range(1, TOPK):
            s = s + sel_w[r]
        inv = 1.0 / s
        for r in range(TOPK):
            pes_v = plsc.load_gather(pesbuf, [sel_i[r]])
            plsc.store_scatter(wbuf, [outbase + r], sel_w[r] * inv * pes_v)
            plsc.store_scatter(ibuf, [outbase + r], sel_i[r])
        return carry

    lax.fori_loop(0, tpw // LANES, group_body, 0)

    for c in range(EXPERTS // LANES):
        acc = fzero
        for l in range(LANES):
            acc = acc + hbuf[pl.ds(l * EXPERTS + c * LANES, LANES)]
        cntbuf[pl.ds(c * LANES, LANES)] = acc

    pltpu.sync_copy(wbuf, w_hbm.at[pl.ds(base * TOPK, tpw * TOPK)])
    pltpu.sync_copy(ibuf, i_hbm.at[pl.ds(base * TOPK, tpw * TOPK)])
    pltpu.sync_copy(cntbuf, c_hbm.at[wid])


CHUNKS = (2, 2, 2, 1, 1)


@jax.jit
def kernel(hidden_states, W, scale, per_expert_scale):
    tokens = hidden_states.shape[0]
    sv = (scale * (HIDDEN ** -0.5)).reshape(1, HIDDEN)
    wt = W.T

    def probs_call(off, nblk, hs, wt, sv):
        return pl.pallas_call(
            _probs_block,
            grid=(nblk,),
            in_specs=[
                pl.BlockSpec((BLOCK, HIDDEN), lambda i: (i + off, 0)),
                pl.BlockSpec((HIDDEN, EXPERTS), lambda i: (0, 0)),
                pl.BlockSpec((1, HIDDEN), lambda i: (0, 0)),
            ],
            out_specs=pl.BlockSpec((BLOCK, EXPERTS), lambda i: (i, 0)),
            out_shape=jax.ShapeDtypeStruct((nblk * BLOCK, EXPERTS),
                                           jnp.float32),
            compiler_params=pltpu.CompilerParams(
                dimension_semantics=("arbitrary",),
            ),
        )(hs, wt, sv)

    mesh = plsc.VectorSubcoreMesh(
        core_axis_name="c", subcore_axis_name="s",
        num_cores=NUM_CORES, num_subcores=NUM_SUBCORES)

    routers = {}

    def router(ctokens):
        tpw = ctokens // NW
        if tpw not in routers:
            routers[tpw] = pl.kernel(
                functools.partial(_sc_router, tpw),
                out_type=[
                    jax.ShapeDtypeStruct((ctokens * TOPK,), jnp.float32),
                    jax.ShapeDtypeStruct((ctokens * TOPK,), jnp.int32),
                    jax.ShapeDtypeStruct((NW, EXPERTS), jnp.float32),
                ],
                mesh=mesh,
                scratch_types=[
                    pltpu.VMEM((tpw * EXPERTS,), jnp.float32),
                    pltpu.VMEM((tpw * TOPK,), jnp.float32),
                    pltpu.VMEM((tpw * TOPK,), jnp.int32),
                    pltpu.VMEM((LANES * EXPERTS,), jnp.float32),
                    pltpu.VMEM((EXPERTS,), jnp.float32),
                    pltpu.VMEM((EXPERTS,), jnp.float32),
                ],
                compiler_params=pltpu.CompilerParams(
                    needs_layout_passes=False),
            )
        return routers[tpw]

    wouts, iouts, cparts = [], [], []
    off = 0
    for nblk in CHUNKS:
        ctokens = nblk * BLOCK
        probs = probs_call(off, nblk, hidden_states, wt, sv)
        wo, io, cp = router(ctokens)(probs.reshape(ctokens * EXPERTS),
                                     per_expert_scale)
        wouts.append(wo.reshape(ctokens, TOPK))
        iouts.append(io.reshape(ctokens, TOPK))
        cparts.append(cp)
        off += nblk
    cnt = cparts[0]
    for c in range(1, len(cparts)):
        cnt = cnt + cparts[c]
    return (jnp.concatenate(wouts, axis=0), jnp.concatenate(iouts, axis=0),
            jnp.sum(cnt, axis=0))


# speedup vs baseline: 1.0691x; 1.0691x over previous
import functools

import jax
import jax.numpy as jnp
from jax import lax
from jax.experimental import pallas as pl
from jax.experimental.pallas import tpu as pltpu
from jax.experimental.pallas import tpu_sc as plsc

HIDDEN = 2048
EXPERTS = 64
TOPK = 8
EPS = 1e-6
BLOCK = 2048

NUM_CORES = 2
NUM_SUBCORES = 16
LANES = 16
NW = NUM_CORES * NUM_SUBCORES
MASKED = -2.0


def _probs_block(h_ref, w_ref, sv_ref, p_ref):
    h = h_ref[...]
    var = jnp.mean(h * h, axis=1, keepdims=True)
    hn = h * jax.lax.rsqrt(var + EPS)
    hn = hn * sv_ref[...]
    logits = jnp.dot(hn, w_ref[...], preferred_element_type=jnp.float32)
    mx = jnp.max(logits, axis=1, keepdims=True)
    ex = jnp.exp(logits - mx)
    p_ref[...] = ex / jnp.sum(ex, axis=1, keepdims=True)


def _sc_router(tpw, p_hbm, pes_hbm, w_hbm, i_hbm, c_hbm,
               pbuf, wbuf, ibuf, hbuf, pesbuf, cntbuf):
    wid = lax.axis_index("s") * NUM_CORES + lax.axis_index("c")
    base = wid * tpw
    lanes = lax.iota(jnp.int32, LANES)
    fzero = jnp.zeros((LANES,), jnp.float32)
    ones = jnp.ones((LANES,), jnp.float32)

    pltpu.sync_copy(p_hbm.at[pl.ds(base * EXPERTS, tpw * EXPERTS)], pbuf)
    pltpu.sync_copy(pes_hbm, pesbuf)
    for i in range(EXPERTS * LANES // LANES):
        hbuf[pl.ds(i * LANES, LANES)] = fzero

    def merge(am, ai, bm, bi):
        take = am >= bm
        return jnp.where(take, am, bm), jnp.where(take, ai, bi)

    def chunk_tree(idx_base, const_off):
        ms, is_ = [], []
        for j in range(0, LANES, 2):
            va = plsc.load_gather(pbuf, [idx_base + j])
            vb = plsc.load_gather(pbuf, [idx_base + (j + 1)])
            take = va >= vb
            ms.append(jnp.where(take, va, vb))
            is_.append(jnp.where(take,
                                 jnp.full((LANES,), j, jnp.int32),
                                 jnp.full((LANES,), j + 1, jnp.int32)))
        while len(ms) > 1:
            nm, ni = [], []
            for a in range(0, len(ms), 2):
                m2, i2 = merge(ms[a], is_[a], ms[a + 1], is_[a + 1])
                nm.append(m2)
                ni.append(i2)
            ms, is_ = nm, ni
        return ms[0], is_[0] + const_off

    def group_body(g, carry):
        rows = g * LANES + lanes
        rowbase = rows * EXPERTS
        outbase = rows * TOPK

        nchunks = EXPERTS // LANES
        cms, cis = [], []
        for c in range(nchunks):
            m, i = chunk_tree(rowbase + c * LANES,
                              jnp.full((LANES,), c * LANES, jnp.int32))
            cms.append(m)
            cis.append(i)

        def merge4():
            m01, i01 = merge(cms[0], cis[0], cms[1], cis[1])
            m23, i23 = merge(cms[2], cis[2], cms[3], cis[3])
            return merge(m01, i01, m23, i23)

        gm, gi = merge4()
        sel_w = []
        sel_i = []
        for r in range(TOPK):
            sel_w.append(gm)
            sel_i.append(gi)
            plsc.store_scatter(pbuf, [rowbase + gi],
                               jnp.full((LANES,), MASKED, jnp.float32))
            plsc.addupdate_scatter(hbuf, [lanes * EXPERTS + gi], ones)
            if r < TOPK - 1:
                cstar = jnp.right_shift(gi, 4)
                coff = jnp.left_shift(cstar, 4)
                nm, ni = chunk_tree(rowbase + coff, coff)
                for c in range(nchunks):
                    hit = cstar == c
                    cms[c] = jnp.where(hit, nm, cms[c])
                    cis[c] = jnp.where(hit, ni, cis[c])
                gm, gi = merge4()
        s = sel_w[0]
        for r in range(1, TOPK):
            s = s + sel_w[r]
        inv = 1.0 / s
        for r in range(TOPK):
            pes_v = plsc.load_gather(pesbuf, [sel_i[r]])
            plsc.store_scatter(wbuf, [outbase + r], sel_w[r] * inv * pes_v)
            plsc.store_scatter(ibuf, [outbase + r], sel_i[r])
        return carry

    lax.fori_loop(0, tpw // LANES, group_body, 0)

    for c in range(EXPERTS // LANES):
        acc = fzero
        for l in range(LANES):
            acc = acc + hbuf[pl.ds(l * EXPERTS + c * LANES, LANES)]
        cntbuf[pl.ds(c * LANES, LANES)] = acc

    pltpu.sync_copy(wbuf, w_hbm.at[pl.ds(base * TOPK, tpw * TOPK)])
    pltpu.sync_copy(ibuf, i_hbm.at[pl.ds(base * TOPK, tpw * TOPK)])
    pltpu.sync_copy(cntbuf, c_hbm.at[wid])


CHUNKS = (2, 2, 2, 2)


@jax.jit
def kernel(hidden_states, W, scale, per_expert_scale):
    tokens = hidden_states.shape[0]
    sv = (scale * (HIDDEN ** -0.5)).reshape(1, HIDDEN)
    wt = W.T

    def probs_call(off, nblk, hs, wt, sv):
        return pl.pallas_call(
            _probs_block,
            grid=(nblk,),
            in_specs=[
                pl.BlockSpec((BLOCK, HIDDEN), lambda i: (i + off, 0)),
                pl.BlockSpec((HIDDEN, EXPERTS), lambda i: (0, 0)),
                pl.BlockSpec((1, HIDDEN), lambda i: (0, 0)),
            ],
            out_specs=pl.BlockSpec((BLOCK, EXPERTS), lambda i: (i, 0)),
            out_shape=jax.ShapeDtypeStruct((nblk * BLOCK, EXPERTS),
                                           jnp.float32),
            compiler_params=pltpu.CompilerParams(
                dimension_semantics=("arbitrary",),
            ),
        )(hs, wt, sv)

    mesh = plsc.VectorSubcoreMesh(
        core_axis_name="c", subcore_axis_name="s",
        num_cores=NUM_CORES, num_subcores=NUM_SUBCORES)

    routers = {}

    def router(ctokens):
        tpw = ctokens // NW
        if tpw not in routers:
            routers[tpw] = pl.kernel(
                functools.partial(_sc_router, tpw),
                out_type=[
                    jax.ShapeDtypeStruct((ctokens * TOPK,), jnp.float32),
                    jax.ShapeDtypeStruct((ctokens * TOPK,), jnp.int32),
                    jax.ShapeDtypeStruct((NW, EXPERTS), jnp.float32),
                ],
                mesh=mesh,
                scratch_types=[
                    pltpu.VMEM((tpw * EXPERTS,), jnp.float32),
                    pltpu.VMEM((tpw * TOPK,), jnp.float32),
                    pltpu.VMEM((tpw * TOPK,), jnp.int32),
                    pltpu.VMEM((LANES * EXPERTS,), jnp.float32),
                    pltpu.VMEM((EXPERTS,), jnp.float32),
                    pltpu.VMEM((EXPERTS,), jnp.float32),
                ],
                compiler_params=pltpu.CompilerParams(
                    needs_layout_passes=False),
            )
        return routers[tpw]

    wouts, iouts, cparts = [], [], []
    off = 0
    for nblk in CHUNKS:
        ctokens = nblk * BLOCK
        probs = probs_call(off, nblk, hidden_states, wt, sv)
        wo, io, cp = router(ctokens)(probs.reshape(ctokens * EXPERTS),
                                     per_expert_scale)
        wouts.append(wo.reshape(ctokens, TOPK))
        iouts.append(io.reshape(ctokens, TOPK))
        cparts.append(cp)
        off += nblk
    cnt = cparts[0]
    for c in range(1, len(cparts)):
        cnt = cnt + cparts[c]
    return (jnp.concatenate(wouts, axis=0), jnp.concatenate(iouts, axis=0),
            jnp.sum(cnt, axis=0))


# speedup vs baseline: 1.0794x; 1.0096x over previous
import functools

import jax
import jax.numpy as jnp
from jax import lax
from jax.experimental import pallas as pl
from jax.experimental.pallas import tpu as pltpu
from jax.experimental.pallas import tpu_sc as plsc

HIDDEN = 2048
EXPERTS = 64
TOPK = 8
EPS = 1e-6
BLOCK = 2048

NUM_CORES = 2
NUM_SUBCORES = 16
LANES = 16
NW = NUM_CORES * NUM_SUBCORES
MASKED = -2.0


def _probs_block(h_ref, w_ref, sv_ref, p_ref):
    h = h_ref[...]
    var = jnp.mean(h * h, axis=1, keepdims=True)
    hn = h * jax.lax.rsqrt(var + EPS)
    hn = (hn * sv_ref[...]) * (HIDDEN ** -0.5)
    logits = jnp.dot(hn, w_ref[...], preferred_element_type=jnp.float32)
    mx = jnp.max(logits, axis=1, keepdims=True)
    ex = jnp.exp(logits - mx)
    p_ref[...] = ex / jnp.sum(ex, axis=1, keepdims=True)


def _sc_router(tpw, p_hbm, pes_hbm, w_hbm, i_hbm, c_hbm,
               pbuf, wbuf, ibuf, hbuf, pesbuf, cntbuf):
    wid = lax.axis_index("s") * NUM_CORES + lax.axis_index("c")
    base = wid * tpw
    lanes = lax.iota(jnp.int32, LANES)
    fzero = jnp.zeros((LANES,), jnp.float32)
    ones = jnp.ones((LANES,), jnp.float32)

    pltpu.sync_copy(p_hbm.at[pl.ds(base * EXPERTS, tpw * EXPERTS)], pbuf)
    pltpu.sync_copy(pes_hbm, pesbuf)
    for i in range(EXPERTS * LANES // LANES):
        hbuf[pl.ds(i * LANES, LANES)] = fzero

    def merge(am, ai, bm, bi):
        take = am >= bm
        return jnp.where(take, am, bm), jnp.where(take, ai, bi)

    def chunk_tree(idx_base, const_off):
        ms, is_ = [], []
        for j in range(0, LANES, 2):
            va = plsc.load_gather(pbuf, [idx_base + j])
            vb = plsc.load_gather(pbuf, [idx_base + (j + 1)])
            take = va >= vb
            ms.append(jnp.where(take, va, vb))
            is_.append(jnp.where(take,
                                 jnp.full((LANES,), j, jnp.int32),
                                 jnp.full((LANES,), j + 1, jnp.int32)))
        while len(ms) > 1:
            nm, ni = [], []
            for a in range(0, len(ms), 2):
                m2, i2 = merge(ms[a], is_[a], ms[a + 1], is_[a + 1])
                nm.append(m2)
                ni.append(i2)
            ms, is_ = nm, ni
        return ms[0], is_[0] + const_off

    def group_body(g, carry):
        rows = g * LANES + lanes
        rowbase = rows * EXPERTS
        outbase = rows * TOPK

        nchunks = EXPERTS // LANES
        cms, cis = [], []
        for c in range(nchunks):
            m, i = chunk_tree(rowbase + c * LANES,
                              jnp.full((LANES,), c * LANES, jnp.int32))
            cms.append(m)
            cis.append(i)

        def merge4():
            m01, i01 = merge(cms[0], cis[0], cms[1], cis[1])
            m23, i23 = merge(cms[2], cis[2], cms[3], cis[3])
            return merge(m01, i01, m23, i23)

        gm, gi = merge4()
        sel_w = []
        sel_i = []
        for r in range(TOPK):
            sel_w.append(gm)
            sel_i.append(gi)
            plsc.store_scatter(pbuf, [rowbase + gi],
                               jnp.full((LANES,), MASKED, jnp.float32))
            plsc.addupdate_scatter(hbuf, [lanes * EXPERTS + gi], ones)
            if r < TOPK - 1:
                cstar = jnp.right_shift(gi, 4)
                coff = jnp.left_shift(cstar, 4)
                nm, ni = chunk_tree(rowbase + coff, coff)
                for c in range(nchunks):
                    hit = cstar == c
                    cms[c] = jnp.where(hit, nm, cms[c])
                    cis[c] = jnp.where(hit, ni, cis[c])
                gm, gi = merge4()
        s = sel_w[0]
        for r in range(1, TOPK):
            s = s + sel_w[r]
        inv = 1.0 / s
        for r in range(TOPK):
            pes_v = plsc.load_gather(pesbuf, [sel_i[r]])
            plsc.store_scatter(wbuf, [outbase + r], sel_w[r] * inv * pes_v)
            plsc.store_scatter(ibuf, [outbase + r], sel_i[r])
        return carry

    lax.fori_loop(0, tpw // LANES, group_body, 0)

    for c in range(EXPERTS // LANES):
        acc = fzero
        for l in range(LANES):
            acc = acc + hbuf[pl.ds(l * EXPERTS + c * LANES, LANES)]
        cntbuf[pl.ds(c * LANES, LANES)] = acc

    pltpu.sync_copy(wbuf, w_hbm.at[pl.ds(base * TOPK, tpw * TOPK)])
    pltpu.sync_copy(ibuf, i_hbm.at[pl.ds(base * TOPK, tpw * TOPK)])
    pltpu.sync_copy(cntbuf, c_hbm.at[wid])


CHUNKS = (2, 2, 2, 2)


@jax.jit
def kernel(hidden_states, W, scale, per_expert_scale):
    tokens = hidden_states.shape[0]
    sv = scale.reshape(1, HIDDEN)
    wt = W.T

    def probs_call(off, nblk, hs, wt, sv):
        return pl.pallas_call(
            _probs_block,
            grid=(nblk,),
            in_specs=[
                pl.BlockSpec((BLOCK, HIDDEN), lambda i: (i + off, 0)),
                pl.BlockSpec((HIDDEN, EXPERTS), lambda i: (0, 0)),
                pl.BlockSpec((1, HIDDEN), lambda i: (0, 0)),
            ],
            out_specs=pl.BlockSpec((BLOCK, EXPERTS), lambda i: (i, 0)),
            out_shape=jax.ShapeDtypeStruct((nblk * BLOCK, EXPERTS),
                                           jnp.float32),
            compiler_params=pltpu.CompilerParams(
                dimension_semantics=("arbitrary",),
            ),
        )(hs, wt, sv)

    mesh = plsc.VectorSubcoreMesh(
        core_axis_name="c", subcore_axis_name="s",
        num_cores=NUM_CORES, num_subcores=NUM_SUBCORES)

    routers = {}

    def router(ctokens):
        tpw = ctokens // NW
        if tpw not in routers:
            routers[tpw] = pl.kernel(
                functools.partial(_sc_router, tpw),
                out_type=[
                    jax.ShapeDtypeStruct((ctokens * TOPK,), jnp.float32),
                    jax.ShapeDtypeStruct((ctokens * TOPK,), jnp.int32),
                    jax.ShapeDtypeStruct((NW, EXPERTS), jnp.float32),
                ],
                mesh=mesh,
                scratch_types=[
                    pltpu.VMEM((tpw * EXPERTS,), jnp.float32),
                    pltpu.VMEM((tpw * TOPK,), jnp.float32),
                    pltpu.VMEM((tpw * TOPK,), jnp.int32),
                    pltpu.VMEM((LANES * EXPERTS,), jnp.float32),
                    pltpu.VMEM((EXPERTS,), jnp.float32),
                    pltpu.VMEM((EXPERTS,), jnp.float32),
                ],
                compiler_params=pltpu.CompilerParams(
                    needs_layout_passes=False),
            )
        return routers[tpw]

    wouts, iouts, cparts = [], [], []
    off = 0
    for nblk in CHUNKS:
        ctokens = nblk * BLOCK
        probs = probs_call(off, nblk, hidden_states, wt, sv)
        wo, io, cp = router(ctokens)(probs.reshape(ctokens * EXPERTS),
                                     per_expert_scale)
        wouts.append(wo.reshape(ctokens, TOPK))
        iouts.append(io.reshape(ctokens, TOPK))
        cparts.append(cp)
        off += nblk
    cnt = cparts[0]
    for c in range(1, len(cparts)):
        cnt = cnt + cparts[c]
    return (jnp.concatenate(wouts, axis=0), jnp.concatenate(iouts, axis=0),
            jnp.sum(cnt, axis=0))
